# baseline (speedup 1.0000x reference)
import functools

import jax
import jax.numpy as jnp
from jax import lax
from jax.experimental import pallas as pl
from jax.experimental.pallas import tpu as pltpu
from jax.experimental.pallas import tpu_sc as plsc

D = 32
OUT = 64
BB = 512


@functools.partial(jax.jit, static_argnums=(2, 3))
def _sc_gather(table, idx, n, d):
    info = plsc.get_sparse_core_info()
    nw = info.num_cores * info.num_subcores
    nc = info.num_cores
    n_per_w = n // nw
    k = 16
    mega = 640
    n_mega = n_per_w // mega
    n_bat = mega // k
    mesh = plsc.VectorSubcoreMesh(core_axis_name="c", subcore_axis_name="s")

    @functools.partial(
        pl.kernel,
        mesh=mesh,
        out_type=jax.ShapeDtypeStruct((n, d), jnp.float32),
        scratch_types=[
            pltpu.VMEM((n_per_w,), jnp.int32),
            pltpu.VMEM((mega, d), jnp.float32),
            pltpu.SemaphoreType.DMA,
        ],
        compiler_params=pltpu.CompilerParams(use_tc_tiling_on_sc=True),
    )
    def gather_kernel(table_hbm, idx_hbm, out_hbm, idx_v, buf_v, gsem):
        wid = lax.axis_index("s") * nc + lax.axis_index("c")
        base = wid * n_per_w
        pltpu.sync_copy(idx_hbm.at[pl.ds(base, n_per_w)], idx_v)

        def drain_batch():
            for _ in range(k):
                pltpu.make_async_copy(
                    table_hbm.at[pl.ds(0, 1)],
                    buf_v.at[pl.ds(0, 1)],
                    gsem).wait()

        for m in range(n_mega):
            def body(it, _, m=m):
                @pl.when(it < n_bat)
                def _():
                    j0 = it * k
                    ivec = idx_v[pl.ds(m * mega + j0, 16)]
                    for kk in range(k):
                        pltpu.async_copy(
                            table_hbm.at[pl.ds(ivec[kk], 1)],
                            buf_v.at[pl.ds(j0 + kk, 1)],
                            gsem)

                @pl.when(it > 0)
                def _():
                    drain_batch()
                return 0

            lax.fori_loop(0, n_bat + 1, body, 0)
            pltpu.sync_copy(buf_v, out_hbm.at[pl.ds(base + m * mega, mega)])

    return gather_kernel(table, idx)


def _gru_body(emb_ref, w_ref, bias_ref, wo_ref, bo_ref, out_ref, hseq_ref,
              *, seq_len):
    w = w_ref[...]
    bias = bias_ref[...]
    h = jnp.zeros((BB, 2 * D), jnp.float32)
    for i in range(seq_len):
        xf = emb_ref[:, D * i : D * (i + 1)]
        xb = emb_ref[:, D * (seq_len - 1 - i) : D * (seq_len - i)]
        x = jnp.concatenate([xf, xb, h], axis=-1)
        g = jnp.dot(x, w, preferred_element_type=jnp.float32) + bias
        z = jax.nn.sigmoid(g[:, : 2 * D])
        r = jax.nn.sigmoid(g[:, 2 * D : 4 * D])
        c = jnp.tanh(g[:, 4 * D : 6 * D] + r * g[:, 6 * D :])
        h = z * h + (1.0 - z) * c
        hseq_ref[i, :, :D] = h[:, :D]
        hseq_ref[seq_len - 1 - i, :, D:] = h[:, D:]
    wo = wo_ref[...]
    bo = bo_ref[...]
    for t in range(seq_len):
        o = jnp.dot(hseq_ref[t], wo, preferred_element_type=jnp.float32) + bo
        out_ref[t] = o.T


def _gru_dense(emb, w, bias, wo, bo, seq_len):
    b = emb.shape[0]
    grid = (b // BB,)
    return pl.pallas_call(
        functools.partial(_gru_body, seq_len=seq_len),
        grid=grid,
        in_specs=[
            pl.BlockSpec((BB, seq_len * D), lambda i: (i, 0)),
            pl.BlockSpec((4 * D, 8 * D), lambda i: (0, 0)),
            pl.BlockSpec((1, 8 * D), lambda i: (0, 0)),
            pl.BlockSpec((2 * D, OUT), lambda i: (0, 0)),
            pl.BlockSpec((1, OUT), lambda i: (0, 0)),
        ],
        out_specs=pl.BlockSpec((seq_len, OUT, BB), lambda i: (0, 0, i)),
        out_shape=jax.ShapeDtypeStruct((seq_len, OUT, b), jnp.float32),
        scratch_shapes=[
            pltpu.VMEM((seq_len, BB, 2 * D), jnp.float32),
        ],
    )(emb, w, bias, wo, bo)


def _build_step_weights(K_f, RK_f, b_f, K_b, RK_b, b_b):
    Z = jnp.zeros((D, D), jnp.float32)

    def quad(g, fw, bw):
        c0 = fw[:, g * D : (g + 1) * D]
        c1 = bw[:, g * D : (g + 1) * D]
        return jnp.block([[c0, Z], [Z, c1]])

    rows_xf_xb = jnp.concatenate(
        [quad(0, K_f, K_b), quad(1, K_f, K_b), quad(2, K_f, K_b),
         jnp.zeros((2 * D, 2 * D), jnp.float32)], axis=1)
    rows_hf_hb = jnp.concatenate(
        [quad(0, RK_f, RK_b), quad(1, RK_f, RK_b),
         jnp.zeros((2 * D, 2 * D), jnp.float32), quad(2, RK_f, RK_b)], axis=1)
    w = jnp.concatenate([rows_xf_xb, rows_hf_hb], axis=0)

    def bpair(g, src_f, src_b):
        return jnp.concatenate(
            [src_f[g * D : (g + 1) * D], src_b[g * D : (g + 1) * D]])

    bias = jnp.concatenate([
        bpair(0, b_f[0] + b_f[1], b_b[0] + b_b[1]),
        bpair(1, b_f[0] + b_f[1], b_b[0] + b_b[1]),
        bpair(2, b_f[0], b_b[0]),
        bpair(2, b_f[1], b_b[1]),
    ]).reshape(1, 8 * D)
    return w, bias


def kernel(inputs, table, K_f, RK_f, b_f, K_b, RK_b, b_b, W_out, b_out):
    b, seq_len = inputs.shape
    n = b * seq_len
    idx = inputs.reshape(n).astype(jnp.int32)
    emb_flat = _sc_gather(table, idx, n, D)
    emb = emb_flat.reshape(b, seq_len * D)

    w, bias = _build_step_weights(K_f, RK_f, b_f, K_b, RK_b, b_b)
    bo = b_out.reshape(1, OUT)
    out_p = _gru_dense(emb, w, bias, W_out, bo, seq_len)
    return jnp.transpose(out_p, (2, 0, 1))


# speedup vs baseline: 1.0548x; 1.0548x over previous
import functools

import jax
import jax.numpy as jnp
from jax import lax
from jax.experimental import pallas as pl
from jax.experimental.pallas import tpu as pltpu
from jax.experimental.pallas import tpu_sc as plsc

D = 32
OUT = 64
BB = 1024


@functools.partial(jax.jit, static_argnums=(2, 3))
def _sc_gather(table, idx, n, d):
    info = plsc.get_sparse_core_info()
    nw = info.num_cores * info.num_subcores
    nc = info.num_cores
    n_per_w = n // nw
    k = 32
    mega = 640
    n_mega = n_per_w // mega
    n_bat = mega // k
    mesh = plsc.VectorSubcoreMesh(core_axis_name="c", subcore_axis_name="s")

    @functools.partial(
        pl.kernel,
        mesh=mesh,
        out_type=jax.ShapeDtypeStruct((n, d), jnp.float32),
        scratch_types=[
            pltpu.VMEM((n_per_w,), jnp.int32),
            pltpu.VMEM((mega, d), jnp.float32),
            pltpu.SemaphoreType.DMA,
        ],
        compiler_params=pltpu.CompilerParams(use_tc_tiling_on_sc=True),
    )
    def gather_kernel(table_hbm, idx_hbm, out_hbm, idx_v, buf_v, gsem):
        wid = lax.axis_index("s") * nc + lax.axis_index("c")
        base = wid * n_per_w
        pltpu.sync_copy(idx_hbm.at[pl.ds(base, n_per_w)], idx_v)

        def drain_batch():
            for _ in range(k):
                pltpu.make_async_copy(
                    table_hbm.at[pl.ds(0, 1)],
                    buf_v.at[pl.ds(0, 1)],
                    gsem).wait()

        for m in range(n_mega):
            def body(it, _, m=m):
                @pl.when(it < n_bat)
                def _():
                    j0 = it * k
                    ivecs = [idx_v[pl.ds(m * mega + j0 + 16 * q, 16)]
                             for q in range(k // 16)]
                    for kk in range(k):
                        pltpu.async_copy(
                            table_hbm.at[pl.ds(ivecs[kk // 16][kk % 16], 1)],
                            buf_v.at[pl.ds(j0 + kk, 1)],
                            gsem)

                @pl.when(it > 0)
                def _():
                    drain_batch()
                return 0

            lax.fori_loop(0, n_bat + 1, body, 0)
            pltpu.sync_copy(buf_v, out_hbm.at[pl.ds(base + m * mega, mega)])

    return gather_kernel(table, idx)


def _gru_body(emb_ref, w_ref, bias_ref, wo_ref, bo_ref, out_ref, hseq_ref,
              *, seq_len):
    w = w_ref[...]
    bias = bias_ref[...]
    h = jnp.zeros((BB, 2 * D), jnp.float32)
    for i in range(seq_len):
        xf = emb_ref[:, D * i : D * (i + 1)]
        xb = emb_ref[:, D * (seq_len - 1 - i) : D * (seq_len - i)]
        x = jnp.concatenate([xf, xb, h], axis=-1)
        g = jnp.dot(x, w, preferred_element_type=jnp.float32) + bias
        z = jax.nn.sigmoid(g[:, : 2 * D])
        r = jax.nn.sigmoid(g[:, 2 * D : 4 * D])
        c = jnp.tanh(g[:, 4 * D : 6 * D] + r * g[:, 6 * D :])
        h = z * h + (1.0 - z) * c
        hseq_ref[i, :, :D] = h[:, :D]
        hseq_ref[seq_len - 1 - i, :, D:] = h[:, D:]
    wo = wo_ref[...]
    bo = bo_ref[...]
    for t in range(seq_len):
        o = jnp.dot(hseq_ref[t], wo, preferred_element_type=jnp.float32) + bo
        out_ref[t] = o.T


def _gru_dense(emb, w, bias, wo, bo, seq_len):
    b = emb.shape[0]
    grid = (b // BB,)
    return pl.pallas_call(
        functools.partial(_gru_body, seq_len=seq_len),
        grid=grid,
        in_specs=[
            pl.BlockSpec((BB, seq_len * D), lambda i: (i, 0)),
            pl.BlockSpec((4 * D, 8 * D), lambda i: (0, 0)),
            pl.BlockSpec((1, 8 * D), lambda i: (0, 0)),
            pl.BlockSpec((2 * D, OUT), lambda i: (0, 0)),
            pl.BlockSpec((1, OUT), lambda i: (0, 0)),
        ],
        out_specs=pl.BlockSpec((seq_len, OUT, BB), lambda i: (0, 0, i)),
        out_shape=jax.ShapeDtypeStruct((seq_len, OUT, b), jnp.float32),
        scratch_shapes=[
            pltpu.VMEM((seq_len, BB, 2 * D), jnp.float32),
        ],
    )(emb, w, bias, wo, bo)


def _build_step_weights(K_f, RK_f, b_f, K_b, RK_b, b_b):
    Z = jnp.zeros((D, D), jnp.float32)

    def quad(g, fw, bw):
        c0 = fw[:, g * D : (g + 1) * D]
        c1 = bw[:, g * D : (g + 1) * D]
        return jnp.block([[c0, Z], [Z, c1]])

    rows_xf_xb = jnp.concatenate(
        [quad(0, K_f, K_b), quad(1, K_f, K_b), quad(2, K_f, K_b),
         jnp.zeros((2 * D, 2 * D), jnp.float32)], axis=1)
    rows_hf_hb = jnp.concatenate(
        [quad(0, RK_f, RK_b), quad(1, RK_f, RK_b),
         jnp.zeros((2 * D, 2 * D), jnp.float32), quad(2, RK_f, RK_b)], axis=1)
    w = jnp.concatenate([rows_xf_xb, rows_hf_hb], axis=0)

    def bpair(g, src_f, src_b):
        return jnp.concatenate(
            [src_f[g * D : (g + 1) * D], src_b[g * D : (g + 1) * D]])

    bias = jnp.concatenate([
        bpair(0, b_f[0] + b_f[1], b_b[0] + b_b[1]),
        bpair(1, b_f[0] + b_f[1], b_b[0] + b_b[1]),
        bpair(2, b_f[0], b_b[0]),
        bpair(2, b_f[1], b_b[1]),
    ]).reshape(1, 8 * D)
    return w, bias


def kernel(inputs, table, K_f, RK_f, b_f, K_b, RK_b, b_b, W_out, b_out):
    b, seq_len = inputs.shape
    n = b * seq_len
    idx = inputs.reshape(n).astype(jnp.int32)
    emb_flat = _sc_gather(table, idx, n, D)
    emb = emb_flat.reshape(b, seq_len * D)

    w, bias = _build_step_weights(K_f, RK_f, b_f, K_b, RK_b, b_b)
    bo = b_out.reshape(1, OUT)
    out_p = _gru_dense(emb, w, bias, W_out, bo, seq_len)
    return jnp.transpose(out_p, (2, 0, 1))
